# Initial kernel scaffold; baseline (speedup 1.0000x reference)
import jax
import jax.numpy as jnp
from jax.experimental import pallas as pl


def kernel(x, u_weight, v_weight, v_bias):
    raise NotImplementedError("write your pallas kernel here")


# speedup vs baseline: 11.0430x; 11.0430x over previous
import functools

import jax
import jax.numpy as jnp
from jax import lax
from jax.experimental import pallas as pl
from jax.experimental.pallas import tpu as pltpu
from jax.experimental.pallas import tpu_sc as plsc

VOCAB = 1000000
HIDDEN = 32
EMB = 128
BATCH = 16384
HIST = 50
NTOK = BATCH * HIST


_INFO = plsc.get_sparse_core_info()
_NC = _INFO.num_cores
_NS = _INFO.num_subcores
_NW = _NC * _NS
_ROWS_PER_W = NTOK // _NW
_GSTEP = 128
_NSTEP = 8
_CHUNK = _GSTEP * _NSTEP
_NCHUNK = _ROWS_PER_W // _CHUNK


def _sc_gather_body(idx_hbm, table_hbm, h_hbm, idx_v, rows_v, sem):
    wid = lax.axis_index("s") * _NC + lax.axis_index("c")
    base = wid * _ROWS_PER_W

    def chunk(c, carry):
        off = base + c * _CHUNK
        pltpu.sync_copy(idx_hbm.at[pl.ds(off, _CHUNK)], idx_v)
        copies = []
        for j in range(_NSTEP):
            copies.append(pltpu.async_copy(
                table_hbm.at[idx_v.at[pl.ds(j * _GSTEP, _GSTEP)]],
                rows_v.at[pl.ds(j * _GSTEP, _GSTEP)],
                sem,
            ))
        for cp in copies:
            cp.wait()
        pltpu.sync_copy(rows_v, h_hbm.at[pl.ds(off, _CHUNK)])
        return carry

    lax.fori_loop(0, _NCHUNK, chunk, 0)


def _sc_gather(x_flat, u_weight):
    mesh = plsc.VectorSubcoreMesh(core_axis_name="c", subcore_axis_name="s")
    k = pl.kernel(
        _sc_gather_body,
        out_type=jax.ShapeDtypeStruct((NTOK, HIDDEN), jnp.float32),
        mesh=mesh,
        scratch_types=[
            pltpu.VMEM((_CHUNK,), jnp.int32),
            pltpu.VMEM((_CHUNK, HIDDEN), jnp.float32),
            pltpu.SemaphoreType.DMA,
        ],
        compiler_params=pltpu.CompilerParams(use_tc_tiling_on_sc=False),
    )
    return k(x_flat, u_weight)


_RB = 8192


def _mm_body(h_ref, v_ref, b_ref, o_ref):
    o_ref[...] = (
        jnp.dot(h_ref[...], v_ref[...], preferred_element_type=jnp.float32)
        + b_ref[...]
    )


def _tc_project(h, v_weight, v_bias):
    grid = (NTOK // _RB,)
    return pl.pallas_call(
        _mm_body,
        grid=grid,
        in_specs=[
            pl.BlockSpec((_RB, HIDDEN), lambda i: (i, 0)),
            pl.BlockSpec((HIDDEN, EMB), lambda i: (0, 0)),
            pl.BlockSpec((1, EMB), lambda i: (0, 0)),
        ],
        out_specs=pl.BlockSpec((_RB, EMB), lambda i: (i, 0)),
        out_shape=jax.ShapeDtypeStruct((NTOK, EMB), jnp.float32),
    )(h, v_weight, v_bias.reshape(1, EMB))


@jax.jit
def kernel(x, u_weight, v_weight, v_bias):
    x_flat = x.reshape(-1).astype(jnp.int32)
    h = _sc_gather(x_flat, u_weight)
    out = _tc_project(h, v_weight, v_bias)
    return out.reshape(BATCH, HIST, EMB)


# speedup vs baseline: 15.7327x; 1.4247x over previous
import jax
import jax.numpy as jnp
from jax import lax
from jax.experimental import pallas as pl
from jax.experimental.pallas import tpu as pltpu
from jax.experimental.pallas import tpu_sc as plsc

VOCAB = 1000000
HIDDEN = 32
EMB = 128
BATCH = 16384
HIST = 50
NTOK = BATCH * HIST


_WBLK = 5000


def _wb_body(u_ref, v_ref, b_ref, w_ref):
    w_ref[...] = (
        jnp.dot(u_ref[...], v_ref[...], preferred_element_type=jnp.float32)
        + b_ref[...]
    )


def _build_w(u, v, b):
    return pl.pallas_call(
        _wb_body,
        grid=(VOCAB // _WBLK,),
        in_specs=[
            pl.BlockSpec((_WBLK, HIDDEN), lambda i: (i, 0)),
            pl.BlockSpec((HIDDEN, EMB), lambda i: (0, 0)),
            pl.BlockSpec((1, EMB), lambda i: (0, 0)),
        ],
        out_specs=pl.BlockSpec((_WBLK, EMB), lambda i: (i, 0)),
        out_shape=jax.ShapeDtypeStruct((VOCAB, EMB), jnp.float32),
    )(u, v, b.reshape(1, EMB))


_INFO = plsc.get_sparse_core_info()
_NC = _INFO.num_cores
_NS = _INFO.num_subcores
_NW = _NC * _NS
_BATCH_PER_W = BATCH // _NW
_CB = 8
_CTOK = _CB * HIST
_NCHUNK = _BATCH_PER_W // _CB
_GATHERS = ((0, 128), (128, 128), (256, 128), (384, 16))


def _sc_body(idx_hbm, w_hbm, out_hbm, idx_v, rows_v, sem):
    wid = lax.axis_index("s") * _NC + lax.axis_index("c")
    b0 = wid * _BATCH_PER_W

    def chunk(c, carry):
        bb = b0 + c * _CB
        pltpu.sync_copy(idx_hbm.at[pl.ds(bb * HIST, _CTOK)], idx_v)
        copies = []
        for off, n in _GATHERS:
            copies.append(pltpu.async_copy(
                w_hbm.at[idx_v.at[pl.ds(off, n)]],
                rows_v.at[pl.ds(off, n)],
                sem,
            ))
        for cp in copies:
            cp.wait()
        for b in range(_CB):
            pltpu.sync_copy(
                rows_v.at[pl.ds(b * HIST, HIST)],
                out_hbm.at[bb + b],
            )
        return carry

    lax.fori_loop(0, _NCHUNK, chunk, 0)


def _sc_gather_out(x_flat, w):
    mesh = plsc.VectorSubcoreMesh(core_axis_name="c", subcore_axis_name="s")
    k = pl.kernel(
        _sc_body,
        out_type=jax.ShapeDtypeStruct((BATCH, HIST, EMB), jnp.float32),
        mesh=mesh,
        scratch_types=[
            pltpu.VMEM((_CTOK,), jnp.int32),
            pltpu.VMEM((_CTOK, EMB), jnp.float32),
            pltpu.SemaphoreType.DMA,
        ],
        compiler_params=pltpu.CompilerParams(use_tc_tiling_on_sc=True),
    )
    return k(x_flat, w)


@jax.jit
def kernel(x, u_weight, v_weight, v_bias):
    x_flat = x.reshape(-1).astype(jnp.int32)
    w = _build_w(u_weight, v_weight, v_bias)
    return _sc_gather_out(x_flat, w)


# speedup vs baseline: 16.2960x; 1.0358x over previous
import jax
import jax.numpy as jnp
from jax import lax
from jax.experimental import pallas as pl
from jax.experimental.pallas import tpu as pltpu
from jax.experimental.pallas import tpu_sc as plsc

VOCAB = 1000000
HIDDEN = 32
EMB = 128
BATCH = 16384
HIST = 50
NTOK = BATCH * HIST


_WBLK = 20000


def _wb_body(u_ref, v_ref, b_ref, w_ref):
    w_ref[...] = (
        jnp.dot(u_ref[...], v_ref[...], preferred_element_type=jnp.float32)
        + b_ref[...]
    )


def _build_w(u, v, b):
    return pl.pallas_call(
        _wb_body,
        grid=(VOCAB // _WBLK,),
        in_specs=[
            pl.BlockSpec((_WBLK, HIDDEN), lambda i: (i, 0)),
            pl.BlockSpec((HIDDEN, EMB), lambda i: (0, 0)),
            pl.BlockSpec((1, EMB), lambda i: (0, 0)),
        ],
        out_specs=pl.BlockSpec((_WBLK, EMB), lambda i: (i, 0)),
        out_shape=jax.ShapeDtypeStruct((VOCAB, EMB), jnp.float32),
    )(u, v, b.reshape(1, EMB))


_INFO = plsc.get_sparse_core_info()
_NC = _INFO.num_cores
_NS = _INFO.num_subcores
_NW = _NC * _NS
_BATCH_PER_W = BATCH // _NW
_CB = 8
_CTOK = _CB * HIST
_NCHUNK = _BATCH_PER_W // _CB
_GATHERS = ((0, 128), (128, 128), (256, 128), (384, 16))


def _sc_body(idx_hbm, w_hbm, out_hbm, idx_v, rows_v, sem):
    wid = lax.axis_index("s") * _NC + lax.axis_index("c")
    b0 = wid * _BATCH_PER_W

    def chunk(c, carry):
        bb = b0 + c * _CB
        pltpu.sync_copy(idx_hbm.at[pl.ds(bb * HIST, _CTOK)], idx_v)
        copies = []
        for off, n in _GATHERS:
            copies.append(pltpu.async_copy(
                w_hbm.at[idx_v.at[pl.ds(off, n)]],
                rows_v.at[pl.ds(off, n)],
                sem,
            ))
        for cp in copies:
            cp.wait()
        for b in range(_CB):
            pltpu.sync_copy(
                rows_v.at[pl.ds(b * HIST, HIST)],
                out_hbm.at[bb + b],
            )
        return carry

    lax.fori_loop(0, _NCHUNK, chunk, 0)


def _sc_gather_out(x_flat, w):
    mesh = plsc.VectorSubcoreMesh(core_axis_name="c", subcore_axis_name="s")
    k = pl.kernel(
        _sc_body,
        out_type=jax.ShapeDtypeStruct((BATCH, HIST, EMB), jnp.float32),
        mesh=mesh,
        scratch_types=[
            pltpu.VMEM((_CTOK,), jnp.int32),
            pltpu.VMEM((_CTOK, EMB), jnp.float32),
            pltpu.SemaphoreType.DMA,
        ],
        compiler_params=pltpu.CompilerParams(use_tc_tiling_on_sc=True),
    )
    return k(x_flat, w)


@jax.jit
def kernel(x, u_weight, v_weight, v_bias):
    x_flat = x.reshape(-1).astype(jnp.int32)
    w = _build_w(u_weight, v_weight, v_bias)
    return _sc_gather_out(x_flat, w)
